# baseline (speedup 1.0000x reference)
import functools

import jax
import jax.numpy as jnp
from jax import lax
from jax.experimental import pallas as pl
from jax.experimental.pallas import tpu as pltpu
from jax.experimental.pallas import tpu_sc as plsc

BATCH = 4096
N_K = 8192
BT = 512

SC_NC = 2
SC_NS = 16
SC_L = 16
SC_NW = SC_NC * SC_NS
B_PER_W = BATCH // SC_NW


def _winner_body(x_ref, wneg2_ref, b2_ref, win_ref):
    x0 = x_ref[:, 0:1]
    x1 = x_ref[:, 1:2]
    radial = jnp.sqrt(x0 * x0 + x1 * x1)
    scale = jnp.maximum(radial, 1.0)
    xs0 = x0 / scale
    xs1 = x1 / scale
    rsq = xs0 * xs0 + xs1 * xs1
    xs2 = jnp.sqrt(jnp.clip(1.0 - rsq, 0.0, 1.0))

    xb = jnp.concatenate(
        [
            xs0.astype(jnp.bfloat16),
            xs1.astype(jnp.bfloat16),
            xs2.astype(jnp.bfloat16),
            jnp.zeros((BT, 5), jnp.bfloat16),
        ],
        axis=1,
    )
    cpp = lax.dot_general(
        xb,
        wneg2_ref[...],
        (((1,), (1,)), ((), ())),
        preferred_element_type=jnp.float32,
    )

    a2 = xs0 * xs0 + xs1 * xs1 + xs2 * xs2
    b2 = b2_ref[...]
    dist = jnp.maximum((a2 + b2) + cpp, 0.0)
    db = lax.bitcast_convert_type(dist, jnp.int32)
    m = jnp.min(db, axis=1, keepdims=True)
    iota = lax.broadcasted_iota(jnp.int32, (BT, N_K), 1)
    win_ref[...] = jnp.min(
        jnp.where(db == m, iota, N_K), axis=1, keepdims=True
    )


def _winners_tc(x, wneg2, b2):
    return pl.pallas_call(
        _winner_body,
        grid=(BATCH // BT,),
        in_specs=[
            pl.BlockSpec((BT, 2), lambda i: (i, 0)),
            pl.BlockSpec((N_K, 8), lambda i: (0, 0)),
            pl.BlockSpec((1, N_K), lambda i: (0, 0)),
        ],
        out_specs=pl.BlockSpec((BT, 1), lambda i: (i, 0)),
        out_shape=jax.ShapeDtypeStruct((BATCH, 1), jnp.int32),
        compiler_params=pltpu.CompilerParams(
            dimension_semantics=("parallel",),
        ),
    )(x, wneg2, b2)


@functools.cache
def _gather_sc():
    @functools.partial(
        pl.kernel,
        mesh=plsc.VectorSubcoreMesh(core_axis_name="c", subcore_axis_name="s"),
        out_type=jax.ShapeDtypeStruct((BATCH,), jnp.float32),
        scratch_types=[
            pltpu.VMEM((B_PER_W,), jnp.int32),
            pltpu.VMEM((B_PER_W,), jnp.float32),
            pltpu.SemaphoreType.DMA,
        ],
    )
    def gather(gw_hbm, idx_hbm, out_hbm, idx_v, out_v, sem):
        wid = lax.axis_index("s") * SC_NC + lax.axis_index("c")
        base = wid * B_PER_W
        pltpu.sync_copy(idx_hbm.at[pl.ds(base, B_PER_W)], idx_v)
        pltpu.async_copy(gw_hbm.at[idx_v], out_v, sem).wait()
        pltpu.sync_copy(out_v, out_hbm.at[pl.ds(base, B_PER_W)])

    return gather


def kernel(x, kohonen_weights, grossberg_weights):
    w_bf = kohonen_weights.astype(jnp.bfloat16)
    wneg2 = (-2.0 * w_bf.astype(jnp.float32)).astype(jnp.bfloat16)
    wneg2 = jnp.concatenate(
        [wneg2, jnp.zeros((N_K, 5), jnp.bfloat16)], axis=1
    )
    b2 = jnp.sum(jnp.square(kohonen_weights), axis=1).reshape(1, N_K)
    winners = _winners_tc(x, wneg2, b2)
    out = _gather_sc()(grossberg_weights.reshape(-1), winners.reshape(-1))
    return out.reshape(BATCH, 1)


# speedup vs baseline: 1.0571x; 1.0571x over previous
import functools

import jax
import jax.numpy as jnp
from jax import lax
from jax.experimental import pallas as pl
from jax.experimental.pallas import tpu as pltpu
from jax.experimental.pallas import tpu_sc as plsc

BATCH = 4096
N_K = 8192
BT = 512

SC_NC = 2
SC_NS = 16
SC_L = 16
SC_NW = SC_NC * SC_NS
B_PER_W = BATCH // SC_NW


def _winner_body(x_ref, wneg2_ref, kwt_ref, win_ref):
    x0 = x_ref[:, 0:1]
    x1 = x_ref[:, 1:2]
    radial = jnp.sqrt(x0 * x0 + x1 * x1)
    scale = jnp.maximum(radial, 1.0)
    xs0 = x0 / scale
    xs1 = x1 / scale
    rsq = xs0 * xs0 + xs1 * xs1
    xs2 = jnp.sqrt(jnp.clip(1.0 - rsq, 0.0, 1.0))

    xb = jnp.concatenate(
        [
            xs0.astype(jnp.bfloat16),
            xs1.astype(jnp.bfloat16),
            xs2.astype(jnp.bfloat16),
            jnp.zeros((BT, 5), jnp.bfloat16),
        ],
        axis=1,
    )
    cpp = lax.dot_general(
        xb,
        wneg2_ref[...],
        (((1,), (0,)), ((), ())),
        preferred_element_type=jnp.float32,
    )

    w0 = kwt_ref[0:1, :]
    w1 = kwt_ref[1:2, :]
    w2 = kwt_ref[2:3, :]
    a2 = xs0 * xs0 + xs1 * xs1 + xs2 * xs2
    b2 = w0 * w0 + w1 * w1 + w2 * w2
    dist = jnp.maximum((a2 + b2) + cpp, 0.0)
    db = lax.bitcast_convert_type(dist, jnp.int32)
    m = jnp.min(db, axis=1, keepdims=True)
    iota = lax.broadcasted_iota(jnp.int32, (BT, N_K), 1)
    win_ref[...] = jnp.min(
        jnp.where(db == m, iota, N_K), axis=1, keepdims=True
    )


def _winners_tc(x, wneg2, kwt):
    return pl.pallas_call(
        _winner_body,
        grid=(BATCH // BT,),
        in_specs=[
            pl.BlockSpec((BT, 2), lambda i: (i, 0)),
            pl.BlockSpec((8, N_K), lambda i: (0, 0)),
            pl.BlockSpec((3, N_K), lambda i: (0, 0)),
        ],
        out_specs=pl.BlockSpec((BT, 1), lambda i: (i, 0)),
        out_shape=jax.ShapeDtypeStruct((BATCH, 1), jnp.int32),
        compiler_params=pltpu.CompilerParams(
            dimension_semantics=("parallel",),
        ),
    )(x, wneg2, kwt)


@functools.cache
def _gather_sc():
    @functools.partial(
        pl.kernel,
        mesh=plsc.VectorSubcoreMesh(core_axis_name="c", subcore_axis_name="s"),
        out_type=jax.ShapeDtypeStruct((BATCH,), jnp.float32),
        scratch_types=[
            pltpu.VMEM((B_PER_W,), jnp.int32),
            pltpu.VMEM((B_PER_W,), jnp.float32),
            pltpu.SemaphoreType.DMA,
        ],
    )
    def gather(gw_hbm, idx_hbm, out_hbm, idx_v, out_v, sem):
        wid = lax.axis_index("s") * SC_NC + lax.axis_index("c")
        base = wid * B_PER_W
        pltpu.sync_copy(idx_hbm.at[pl.ds(base, B_PER_W)], idx_v)
        pltpu.async_copy(gw_hbm.at[idx_v], out_v, sem).wait()
        pltpu.sync_copy(out_v, out_hbm.at[pl.ds(base, B_PER_W)])

    return gather


def kernel(x, kohonen_weights, grossberg_weights):
    w_bf = kohonen_weights.astype(jnp.bfloat16)
    wneg2 = (-2.0 * w_bf.astype(jnp.float32)).astype(jnp.bfloat16)
    wneg2 = jnp.concatenate(
        [wneg2.T, jnp.zeros((5, N_K), jnp.bfloat16)], axis=0
    )
    winners = _winners_tc(x, wneg2, kohonen_weights.T)
    out = _gather_sc()(grossberg_weights.reshape(-1), winners.reshape(-1))
    return out.reshape(BATCH, 1)
